# baseline (speedup 1.0000x reference)
import functools

import jax
import jax.numpy as jnp
from jax.experimental import pallas as pl
from jax.experimental.pallas import tpu as pltpu

_B, _CIN, _N, _CSH = 4, 256, 20000, 64
_HEAD_SPLITS = (2, 1, 3, 2)
_COUT = sum(_HEAD_SPLITS)
_TN = 2048


def _fused_kernel(x_ref, wsh_ref, bsh_ref, w1_ref, b1_ref, w2_ref, b2_ref,
                  out_ref):
    x = x_ref[0]
    y = jnp.dot(wsh_ref[...], x, preferred_element_type=jnp.float32)
    y = jnp.maximum(y + bsh_ref[...], 0.0)
    h = jnp.dot(w1_ref[...], y, preferred_element_type=jnp.float32)
    h = jnp.maximum(h + b1_ref[...], 0.0)
    o = jnp.dot(w2_ref[...], h, preferred_element_type=jnp.float32)
    out_ref[0] = o + b2_ref[...]


@jax.jit
def kernel(ct_feat, W_sh, b_sh, g_sh, bt_sh, W1, b1, g1, bt1,
           W2_reg, b2_reg, W2_height, b2_height, W2_dim, b2_dim,
           W2_rot, b2_rot):
    eps = 1e-3
    inv = 1.0 / jnp.sqrt(1.0 + eps)

    s_sh = g_sh * inv
    wsh_f = s_sh[:, None] * W_sh
    bsh_f = (s_sh * b_sh + bt_sh)[:, None]

    s1 = g1 * inv
    w1_f = (s1[:, :, None] * W1).reshape(4 * _CSH, _CSH)
    b1_f = (s1 * b1 + bt1).reshape(4 * _CSH, 1)

    w2_f = jnp.zeros((_COUT, 4 * _CSH), jnp.float32)
    b2_parts = []
    row = 0
    for i, (w2, b2) in enumerate(((W2_reg, b2_reg), (W2_height, b2_height),
                                  (W2_dim, b2_dim), (W2_rot, b2_rot))):
        c = w2.shape[0]
        w2_f = jax.lax.dynamic_update_slice(w2_f, w2, (row, i * _CSH))
        b2_parts.append(b2)
        row += c
    b2_f = jnp.concatenate(b2_parts)[:, None]

    n_tiles = pl.cdiv(_N, _TN)
    rep = lambda i, j: (0, 0)
    out = pl.pallas_call(
        _fused_kernel,
        grid=(_B, n_tiles),
        in_specs=[
            pl.BlockSpec((1, _CIN, _TN), lambda i, j: (i, 0, j)),
            pl.BlockSpec((_CSH, _CIN), rep),
            pl.BlockSpec((_CSH, 1), rep),
            pl.BlockSpec((4 * _CSH, _CSH), rep),
            pl.BlockSpec((4 * _CSH, 1), rep),
            pl.BlockSpec((_COUT, 4 * _CSH), rep),
            pl.BlockSpec((_COUT, 1), rep),
        ],
        out_specs=pl.BlockSpec((1, _COUT, _TN), lambda i, j: (i, 0, j)),
        out_shape=jax.ShapeDtypeStruct((_B, _COUT, _N), jnp.float32),
        compiler_params=pltpu.CompilerParams(
            dimension_semantics=("parallel", "parallel")),
    )(ct_feat.astype(jnp.float32), wsh_f, bsh_f, w1_f, b1_f, w2_f, b2_f)

    reg = out[:, 0:2, :]
    height = out[:, 2:3, :]
    dim = out[:, 3:6, :]
    rot = out[:, 6:8, :]
    return (reg, height, dim, rot)


# speedup vs baseline: 1.1080x; 1.1080x over previous
import functools

import jax
import jax.numpy as jnp
from jax.experimental import pallas as pl
from jax.experimental.pallas import tpu as pltpu

_B, _CIN, _N, _CSH = 4, 256, 20000, 64
_HEAD_SPLITS = (2, 1, 3, 2)
_COUT = sum(_HEAD_SPLITS)
_TN = 5120


def _fused_kernel(x_ref, wsh_ref, bsh_ref, w1_ref, b1_ref, w2_ref, b2_ref,
                  out_ref):
    x = x_ref[0]
    y = jnp.dot(wsh_ref[...], x, preferred_element_type=jnp.float32)
    y = jnp.maximum(y + bsh_ref[...], 0.0)
    h = jnp.dot(w1_ref[...], y, preferred_element_type=jnp.float32)
    h = jnp.maximum(h + b1_ref[...], 0.0)
    o = jnp.dot(w2_ref[...], h, preferred_element_type=jnp.float32)
    out_ref[0] = o + b2_ref[...]


@jax.jit
def kernel(ct_feat, W_sh, b_sh, g_sh, bt_sh, W1, b1, g1, bt1,
           W2_reg, b2_reg, W2_height, b2_height, W2_dim, b2_dim,
           W2_rot, b2_rot):
    eps = 1e-3
    inv = 1.0 / jnp.sqrt(1.0 + eps)

    s_sh = g_sh * inv
    wsh_f = s_sh[:, None] * W_sh
    bsh_f = (s_sh * b_sh + bt_sh)[:, None]

    s1 = g1 * inv
    w1_f = (s1[:, :, None] * W1).reshape(4 * _CSH, _CSH)
    b1_f = (s1 * b1 + bt1).reshape(4 * _CSH, 1)

    w2_f = jnp.zeros((_COUT, 4 * _CSH), jnp.float32)
    b2_parts = []
    row = 0
    for i, (w2, b2) in enumerate(((W2_reg, b2_reg), (W2_height, b2_height),
                                  (W2_dim, b2_dim), (W2_rot, b2_rot))):
        c = w2.shape[0]
        w2_f = jax.lax.dynamic_update_slice(w2_f, w2, (row, i * _CSH))
        b2_parts.append(b2)
        row += c
    b2_f = jnp.concatenate(b2_parts)[:, None]

    n_tiles = pl.cdiv(_N, _TN)
    rep = lambda i, j: (0, 0)
    out = pl.pallas_call(
        _fused_kernel,
        grid=(_B, n_tiles),
        in_specs=[
            pl.BlockSpec((1, _CIN, _TN), lambda i, j: (i, 0, j)),
            pl.BlockSpec((_CSH, _CIN), rep),
            pl.BlockSpec((_CSH, 1), rep),
            pl.BlockSpec((4 * _CSH, _CSH), rep),
            pl.BlockSpec((4 * _CSH, 1), rep),
            pl.BlockSpec((_COUT, 4 * _CSH), rep),
            pl.BlockSpec((_COUT, 1), rep),
        ],
        out_specs=pl.BlockSpec((1, _COUT, _TN), lambda i, j: (i, 0, j)),
        out_shape=jax.ShapeDtypeStruct((_B, _COUT, _N), jnp.float32),
        compiler_params=pltpu.CompilerParams(
            dimension_semantics=("parallel", "parallel")),
    )(ct_feat.astype(jnp.float32), wsh_f, bsh_f, w1_f, b1_f, w2_f, b2_f)

    reg = out[:, 0:2, :]
    height = out[:, 2:3, :]
    dim = out[:, 3:6, :]
    rot = out[:, 6:8, :]
    return (reg, height, dim, rot)
